# baseline (speedup 1.0000x reference)
import functools

import jax
import jax.numpy as jnp
from jax import lax
from jax.experimental import pallas as pl
from jax.experimental.pallas import tpu as pltpu
from jax.experimental.pallas import tpu_sc as plsc

_D = 64
_L = 16
_BATCH = 4096
_SEQ = 200
_B_TOTAL = _BATCH * _SEQ
_NC = 2
_NS = 16
_NW = _NC * _NS
_B_PER_W = _B_TOTAL // _NW
_ROWS_PER_W = _BATCH // _NW
_CHUNK = _SEQ
_N_PAIRS = _ROWS_PER_W // 2


def _gather_body(table_hbm, idx_hbm, out_hbm, idx_v, bufg0, bufg1,
                 bufw0, bufw1, sg0, sg1, sw0, sw1):
    wid = lax.axis_index("s") * _NC + lax.axis_index("c")
    row0 = wid * _ROWS_PER_W
    bufgs = (bufg0, bufg1)
    bufws = (bufw0, bufw1)
    sgs = (sg0, sg1)
    sws = (sw0, sw1)

    def start_gather(i, b):
        pltpu.async_copy(
            table_hbm.at[idx_v.at[pl.ds(i * _CHUNK, _CHUNK)]], bufgs[b],
            sgs[b])

    def wait_gather(b):
        pltpu.make_async_copy(
            table_hbm.at[idx_v.at[pl.ds(0, _CHUNK)]], bufgs[b], sgs[b]).wait()

    def start_write(i, v):
        pltpu.async_copy(bufws[v], out_hbm.at[row0 + i], sws[v])

    def wait_write(v):
        pltpu.make_async_copy(bufws[v], out_hbm.at[row0], sws[v]).wait()

    def extract(b, v):
        src = bufgs[b]
        dst = bufws[v]

        def ebody(it, carry):
            r0 = it * 8
            for u in range(8):
                r = r0 + u
                for k in range(_D // _L):
                    dst[r, pl.ds(k * _L, _L)] = src[r, pl.ds(k * _L, _L)]
            return carry

        lax.fori_loop(0, _CHUNK // 8, ebody, 0)

    pltpu.sync_copy(idx_hbm.at[wid], idx_v)

    start_gather(0, 0)
    start_gather(1, 1)

    def pair(p, carry):
        for b in range(2):
            i = 2 * p + b
            wait_gather(b)

            @pl.when(p > 0)
            def _():
                wait_write(b)

            extract(b, b)
            start_write(i, b)

            @pl.when(p < _N_PAIRS - 1)
            def _():
                start_gather(i + 2, b)

        return carry

    lax.fori_loop(0, _N_PAIRS, pair, 0)

    for v in range(2):
        wait_write(v)


@jax.jit
def kernel(x, vocab_table):
    mesh = plsc.VectorSubcoreMesh(core_axis_name="c", subcore_axis_name="s")
    gather = functools.partial(
        pl.kernel,
        mesh=mesh,
        out_type=jax.ShapeDtypeStruct((_BATCH, _SEQ, _D), jnp.float32),
        scratch_types=[
            pltpu.VMEM((_B_PER_W,), jnp.int32),
            pltpu.VMEM((_CHUNK, 2 * _D), jnp.float32),
            pltpu.VMEM((_CHUNK, 2 * _D), jnp.float32),
            pltpu.VMEM((_CHUNK, _D), jnp.float32),
            pltpu.VMEM((_CHUNK, _D), jnp.float32),
            pltpu.SemaphoreType.DMA,
            pltpu.SemaphoreType.DMA,
            pltpu.SemaphoreType.DMA,
            pltpu.SemaphoreType.DMA,
        ],
        compiler_params=pltpu.CompilerParams(use_tc_tiling_on_sc=True),
    )(_gather_body)
    table_wide = jnp.concatenate([vocab_table, vocab_table], axis=1)
    return gather(table_wide, x.reshape(_NW, _B_PER_W))


# speedup vs baseline: 1.0244x; 1.0244x over previous
import functools

import jax
import jax.numpy as jnp
from jax import lax
from jax.experimental import pallas as pl
from jax.experimental.pallas import tpu as pltpu
from jax.experimental.pallas import tpu_sc as plsc

_D = 64
_BATCH = 4096
_SEQ = 200
_B_TOTAL = _BATCH * _SEQ
_NC = 2
_NS = 16
_NW = _NC * _NS
_B_PER_W = _B_TOTAL // _NW
_ROWS_PER_W = _BATCH // _NW
_CHUNK = _SEQ
_NBUF = 4
_N_GROUPS = _ROWS_PER_W // _NBUF


def _gather_body(table_hbm, idx_hbm, out_hbm, idx_v, buf0, buf1, buf2, buf3,
                 s0, s1, s2, s3):
    wid = lax.axis_index("s") * _NC + lax.axis_index("c")
    row0 = wid * _ROWS_PER_W
    bufs = (buf0, buf1, buf2, buf3)
    sems = (s0, s1, s2, s3)

    def start_gather(i, b):
        pltpu.async_copy(
            table_hbm.at[idx_v.at[pl.ds(i * _CHUNK, _CHUNK)]], bufs[b],
            sems[b])

    def wait_gather(b):
        pltpu.make_async_copy(
            table_hbm.at[idx_v.at[pl.ds(0, _CHUNK)]], bufs[b], sems[b]).wait()

    def start_write(i, b):
        pltpu.async_copy(bufs[b], out_hbm.at[row0 + i], sems[b])

    def wait_write(b):
        pltpu.make_async_copy(bufs[b], out_hbm.at[row0], sems[b]).wait()

    pltpu.sync_copy(idx_hbm.at[wid], idx_v)

    for b in range(_NBUF):
        start_gather(b, b)

    def group(g, carry):
        for b in range(_NBUF):
            i = g * _NBUF + b
            wait_gather(b)
            start_write(i, b)

        @pl.when(g < _N_GROUPS - 1)
        def _():
            for b in range(_NBUF):
                i = g * _NBUF + b
                wait_write(b)
                start_gather(i + _NBUF, b)

        return carry

    lax.fori_loop(0, _N_GROUPS, group, 0)

    for b in range(_NBUF):
        wait_write(b)


@jax.jit
def kernel(x, vocab_table):
    mesh = plsc.VectorSubcoreMesh(core_axis_name="c", subcore_axis_name="s")
    gather = functools.partial(
        pl.kernel,
        mesh=mesh,
        out_type=jax.ShapeDtypeStruct((_BATCH, _SEQ, _D), jnp.float32),
        scratch_types=[
            pltpu.VMEM((_B_PER_W,), jnp.int32),
            pltpu.VMEM((_CHUNK, _D), jnp.float32),
            pltpu.VMEM((_CHUNK, _D), jnp.float32),
            pltpu.VMEM((_CHUNK, _D), jnp.float32),
            pltpu.VMEM((_CHUNK, _D), jnp.float32),
            pltpu.SemaphoreType.DMA,
            pltpu.SemaphoreType.DMA,
            pltpu.SemaphoreType.DMA,
            pltpu.SemaphoreType.DMA,
        ],
        compiler_params=pltpu.CompilerParams(use_tc_tiling_on_sc=False),
    )(_gather_body)
    return gather(vocab_table, x.reshape(_NW, _B_PER_W))
